# baseline (speedup 1.0000x reference)
import functools

import jax
import jax.numpy as jnp
from jax import lax
from jax.experimental import pallas as pl
from jax.experimental.pallas import tpu as pltpu
from jax.experimental.pallas import tpu_sc as plsc

NUM_CUSTOMERS = 1000000
NUM_ARTICLES = 100000
EMBED_DIM = 32
BATCH = 16384

NC, NS, L = 2, 16, 16
NW = NC * NS
CW = 512
ORD_ROWS = BATCH + 128
NVEC = BATCH // L
FLUSH_AT = 112

_IOTA = lambda: lax.iota(jnp.int32, L)


def _splat(x):
    return jnp.full((L,), x, jnp.int32)


def _make_extract(n_cols, stripe, n_extra, tail_w, fuse_dot):
    nchunk = stripe // CW
    extra0 = NW * stripe
    tail0 = extra0 + n_extra * 128
    assert tail0 + tail_w == n_cols

    def body(*args):
        if fuse_dot:
            (ids_hbm, table_hbm, tail_hbm, aord_hbm, ord_hbm,
             ids_v, ml_id, ml_pos, ring, stag, stag_pos, buf128, tailbuf,
             art_st, sem_ring, sem_st, sem_g) = args
        else:
            (ids_hbm, table_hbm, tail_hbm, ord_hbm,
             ids_v, ml_id, ml_pos, ring, stag, stag_pos, buf128, tailbuf,
             sem_ring, sem_st) = args
        wid = lax.axis_index("s") * NC + lax.axis_index("c")
        lo = wid * stripe
        hi = lo + stripe
        lo2 = jnp.where(wid < n_extra, extra0 + wid * 128,
                        jnp.where(wid == n_extra, tail0, -1))
        hi2 = jnp.where(wid < n_extra, extra0 + wid * 128 + 128,
                        jnp.where(wid == n_extra, tail0 + tail_w, -1))
        dump = BATCH + wid

        pltpu.sync_copy(ids_hbm, ids_v)
        pltpu.sync_copy(tail_hbm, tailbuf)
        pltpu.async_copy(table_hbm.at[:, pl.ds(lo, CW)], ring.at[0], sem_ring)
        pltpu.async_copy(table_hbm.at[:, pl.ds(lo + CW, CW)], ring.at[1],
                         sem_ring)

        def scan_vec(t, cnt):
            v = ids_v[t >> 3, pl.ds((t & 7) * L, L)]
            mask = ((v >= lo) & (v < hi)) | ((v >= lo2) & (v < hi2))
            csum = plsc.cumsum(jnp.where(mask, 1, 0))
            targets = cnt + csum - 1
            plsc.store_scatter(ml_id, [targets], v, mask=mask)
            plsc.store_scatter(ml_pos, [targets], t * L + _IOTA(), mask=mask)
            return cnt + csum[15]

        cnt = lax.fori_loop(0, NVEC, scan_vec, 0)
        nv = (cnt + 15) >> 4

        def pad_and_flush(scnt):
            sl = scnt & 127
            for g in range(8):
                idxv = g * L + _IOTA()
                cur = stag_pos[pl.ds(g * L, L)]
                stag_pos[pl.ds(g * L, L)] = jnp.where(idxv < sl, cur,
                                                      _splat(dump))
            if fuse_dot:
                pltpu.async_copy(aord_hbm.at[stag_pos], art_st, sem_g).wait()

                def dgroup(g, carry):
                    rows = g * L + _IOTA()
                    acc = plsc.load_gather(stag, [rows, _splat(0)]) * \
                        plsc.load_gather(art_st, [rows, _splat(0)])
                    for d in range(1, EMBED_DIM):
                        acc = acc + \
                            plsc.load_gather(stag, [rows, _splat(d)]) * \
                            plsc.load_gather(art_st, [rows, _splat(d)])
                    plsc.store_scatter(art_st, [rows, _splat(0)], acc)
                    return carry

                lax.fori_loop(0, 8, dgroup, 0)
                pltpu.async_copy(art_st, ord_hbm.at[stag_pos],
                                 sem_st).wait()
            else:
                pltpu.async_copy(stag, ord_hbm.at[stag_pos], sem_st).wait()

        def process_range(c0, width, loader, cmask, scnt):
            def per_vec(m, scnt):
                mids = ml_id[pl.ds(m * L, L)]
                valid = (m * L + _IOTA()) < cnt
                mask = (mids >= c0) & (mids < c0 + width) & valid
                k = plsc.all_reduce_population_count(mask)[0]

                def do(s):
                    mpos = ml_pos[pl.ds(m * L, L)]
                    csum = plsc.cumsum(jnp.where(mask, 1, 0))
                    rows = ((s & 127) + csum - 1) & 127
                    cols = (mids - c0) & cmask
                    plsc.store_scatter(stag_pos, [rows], mpos, mask=mask)
                    for d in range(EMBED_DIM):
                        plsc.store_scatter(stag, [rows, _splat(d)],
                                           loader(d, cols), mask=mask)
                    s = s + k

                    def fl(s2):
                        pad_and_flush(s2)
                        return s2 - (s2 & 127) + 128

                    return lax.cond((s & 127) >= FLUSH_AT, fl,
                                    lambda s2: s2, s)

                return lax.cond(k > 0, do, lambda s: s, scnt)

            return lax.fori_loop(0, nv, per_vec, scnt)

        def ring_loader(slot):
            def load(d, cols):
                return plsc.load_gather(ring, [_splat(slot), _splat(d), cols])
            return load

        def buf_loader(buf):
            def load(d, cols):
                return plsc.load_gather(buf, [_splat(d), cols])
            return load

        def chunk(j, scnt):
            pltpu.make_async_copy(table_hbm.at[:, pl.ds(lo, CW)],
                                  ring.at[j & 1], sem_ring).wait()
            scnt = process_range(lo + j * CW, CW, ring_loader(j & 1),
                                 CW - 1, scnt)

            @pl.when(j + 2 < nchunk)
            def _():
                pltpu.async_copy(table_hbm.at[:, pl.ds(lo + (j + 2) * CW, CW)],
                                 ring.at[j & 1], sem_ring)

            return scnt

        scnt = lax.fori_loop(0, nchunk, chunk, 0)

        c128 = jnp.where(wid < n_extra, extra0 + wid * 128, 0)
        pltpu.sync_copy(table_hbm.at[:, pl.ds(c128, 128)], buf128)
        scnt = process_range(c128, jnp.where(wid < n_extra, 128, 0),
                             buf_loader(buf128), 127, scnt)
        scnt = process_range(tail0, tail_w, buf_loader(tailbuf),
                             tail_w - 1, scnt)

        pad_and_flush(scnt)

    return body


def _extract_kernel(n_cols, stripe, n_extra, tail_w, fuse_dot):
    body = _make_extract(n_cols, stripe, n_extra, tail_w, fuse_dot)
    mesh = plsc.VectorSubcoreMesh(core_axis_name="c", subcore_axis_name="s")
    scratch = [
        pltpu.VMEM((BATCH // 128, 128), jnp.int32),
        pltpu.VMEM((BATCH + L,), jnp.int32),
        pltpu.VMEM((BATCH + L,), jnp.int32),
        pltpu.VMEM((2, EMBED_DIM, CW), jnp.float32),
        pltpu.VMEM((128, 128), jnp.float32),
        pltpu.VMEM((128,), jnp.int32),
        pltpu.VMEM((EMBED_DIM, 128), jnp.float32),
        pltpu.VMEM((EMBED_DIM, tail_w), jnp.float32),
    ]
    if fuse_dot:
        scratch += [
            pltpu.VMEM((128, 128), jnp.float32),
        ]
    scratch += [pltpu.SemaphoreType.DMA] * (3 if fuse_dot else 2)
    return functools.partial(
        pl.kernel,
        mesh=mesh,
        out_type=jax.ShapeDtypeStruct((ORD_ROWS, 128), jnp.float32),
        scratch_types=scratch,
        compiler_params=pltpu.CompilerParams(needs_layout_passes=False),
    )(body)


ART_STRIPE = 3072
ART_EXTRA = 13
ART_TAIL = 32
CUST_STRIPE = 31232
CUST_EXTRA = 4
CUST_TAIL = 64


@jax.jit
def _recommend_sc(user2d, article2d, cust_t, cust_tail, art_t, art_tail):
    a_ord = _extract_kernel(NUM_ARTICLES, ART_STRIPE, ART_EXTRA, ART_TAIL,
                            False)(article2d, art_t, art_tail)
    out_wide = _extract_kernel(NUM_CUSTOMERS, CUST_STRIPE, CUST_EXTRA,
                               CUST_TAIL, True)(user2d, cust_t, cust_tail,
                                                a_ord)
    return out_wide[:BATCH, 0]


def kernel(user, article, customer_table, article_table):
    user2d = user.reshape(BATCH // 128, 128)
    article2d = article.reshape(BATCH // 128, 128)
    cust_tail = customer_table[NUM_CUSTOMERS - CUST_TAIL:].T
    art_tail = article_table[NUM_ARTICLES - ART_TAIL:].T
    scores = _recommend_sc(user2d, article2d, customer_table.T, cust_tail,
                           article_table.T, art_tail)
    return scores.reshape(BATCH, 1)


# speedup vs baseline: 1.3321x; 1.3321x over previous
import functools

import jax
import jax.numpy as jnp
from jax import lax
from jax.experimental import pallas as pl
from jax.experimental.pallas import tpu as pltpu
from jax.experimental.pallas import tpu_sc as plsc

NUM_CUSTOMERS = 1000000
NUM_ARTICLES = 100000
EMBED_DIM = 32
BATCH = 16384

NC, NS, L = 2, 16, 16
NW = NC * NS
CW = 256
ORD_ROWS = BATCH + 128
NVEC = BATCH // L

_IOTA = lambda: lax.iota(jnp.int32, L)


def _splat(x):
    return jnp.full((L,), x, jnp.int32)


def _take_ref(ref, i):
    return plsc.load_gather(ref, [_splat(i)])[0]


def _make_extract(n_cols, stripe, n_extra, tail_w):
    nchunk = stripe // CW
    extra0 = NW * stripe
    tail0 = extra0 + n_extra * 128
    assert tail0 + tail_w == n_cols

    def body(ids_hbm, table_hbm, tail_hbm, ord_hbm,
             ids_v, ml_id, ml_pos, ring, stag, stag_pos, tmp_id, tmp_pos,
             buf128, tailbuf, sem_ring, sem_st):
        wid = lax.axis_index("s") * NC + lax.axis_index("c")
        lo = wid * stripe
        hi = lo + stripe
        lo2 = jnp.where(wid < n_extra, extra0 + wid * 128,
                        jnp.where(wid == n_extra, tail0, -1))
        hi2 = jnp.where(wid < n_extra, extra0 + wid * 128 + 128,
                        jnp.where(wid == n_extra, tail0 + tail_w, -1))
        dump = BATCH + wid

        pltpu.sync_copy(ids_hbm, ids_v)
        pltpu.sync_copy(tail_hbm, tailbuf)
        pltpu.async_copy(table_hbm.at[:, pl.ds(lo, CW)], ring.at[0], sem_ring)
        pltpu.async_copy(table_hbm.at[:, pl.ds(lo + CW, CW)], ring.at[1],
                         sem_ring)

        def scan_vec(t, cnt):
            v = ids_v[t >> 3, pl.ds((t & 7) * L, L)]
            mask = ((v >= lo) & (v < hi)) | ((v >= lo2) & (v < hi2))
            k = plsc.all_reduce_population_count(mask)[0]

            def append(c):
                plsc.store_compressed(tmp_id.at[...], v, mask=mask)
                plsc.store_compressed(tmp_pos.at[...], t * L + _IOTA(), mask=mask)
                s0 = c & 15
                c16 = c - s0
                sh = (_IOTA() - s0) & 15
                keep = _IOTA() >= s0
                ml_id[pl.ds(c16, L)] = jnp.where(
                    keep, plsc.load_gather(tmp_id, [sh]),
                    ml_id[pl.ds(c16, L)])
                ml_pos[pl.ds(c16, L)] = jnp.where(
                    keep, plsc.load_gather(tmp_pos, [sh]),
                    ml_pos[pl.ds(c16, L)])
                sh2 = (_IOTA() + 16 - s0) & 15
                ml_id[pl.ds(c16 + L, L)] = plsc.load_gather(tmp_id, [sh2])
                ml_pos[pl.ds(c16 + L, L)] = plsc.load_gather(tmp_pos, [sh2])
                return c + k

            return lax.cond(k > 0, append, lambda c: c, cnt)

        cnt = lax.fori_loop(0, NVEC, scan_vec, 0)
        nv = (cnt + 15) >> 4

        def flush(scnt, posacc):
            s15 = scnt & 15

            @pl.when(s15 == 15)
            def _():
                stag_pos[pl.ds((scnt & 127) - 15, L)] = posacc

            @pl.when((scnt & 127) == 127)
            def _():
                pltpu.async_copy(stag, ord_hbm.at[stag_pos], sem_st).wait()

        def process_range(c0, width, loader, scnt, posacc):
            def per_vec(m, carry):
                scnt, posacc = carry
                mids = ml_id[pl.ds(m * L, L)]
                mpos = ml_pos[pl.ds(m * L, L)]
                valid = (m * L + _IOTA()) < cnt
                mask = jnp.where((mids >= c0) & (mids < c0 + width) & valid,
                                 1, 0)

                def cond(st):
                    return lax.reduce_sum_p.bind(st[0], axes=(0,)) > 0

                def step(st):
                    mask, scnt, posacc = st
                    l = plsc.all_reduce_ffs(mask > 0)[0]
                    cid = _take_ref(ml_id, m * L + l)
                    cpos = _take_ref(ml_pos, m * L + l)
                    g0, g1 = loader(cid - c0)
                    row = scnt & 127
                    stag[row, pl.ds(0, L)] = g0
                    stag[row, pl.ds(L, L)] = g1
                    posacc = jnp.where(_IOTA() == (scnt & 15), cpos, posacc)
                    flush(scnt, posacc)
                    return (jnp.where(_IOTA() == l, 0, mask), scnt + 1,
                            posacc)

                mask, scnt, posacc = lax.while_loop(
                    cond, step, (mask, scnt, posacc))
                return (scnt, posacc)

            return lax.fori_loop(0, nv, per_vec, (scnt, posacc))

        def ring_loader(slot):
            def load(col):
                g0 = plsc.load_gather(ring, [_splat(slot), _IOTA(),
                                             _splat(col)])
                g1 = plsc.load_gather(ring, [_splat(slot), _IOTA() + L,
                                             _splat(col)])
                return g0, g1
            return load

        def buf_loader(buf):
            def load(col):
                g0 = plsc.load_gather(buf, [_IOTA(), _splat(col)])
                g1 = plsc.load_gather(buf, [_IOTA() + L, _splat(col)])
                return g0, g1
            return load

        def chunk(j, carry):
            scnt, posacc = carry
            pltpu.make_async_copy(table_hbm.at[:, pl.ds(lo, CW)],
                                  ring.at[j & 1], sem_ring).wait()
            carry = process_range(lo + j * CW, CW, ring_loader(j & 1),
                                  scnt, posacc)

            @pl.when(j + 2 < nchunk)
            def _():
                pltpu.async_copy(table_hbm.at[:, pl.ds(lo + (j + 2) * CW, CW)],
                                 ring.at[j & 1], sem_ring)

            return carry

        scnt, posacc = lax.fori_loop(0, nchunk, chunk, (0, jnp.zeros(
            (L,), jnp.int32)))

        c128 = jnp.where(wid < n_extra, extra0 + wid * 128, 0)
        pltpu.sync_copy(table_hbm.at[:, pl.ds(c128, 128)], buf128)
        scnt, posacc = process_range(c128, jnp.where(wid < n_extra, 128, 0),
                                     buf_loader(buf128), scnt, posacc)
        scnt, posacc = process_range(tail0, tail_w, buf_loader(tailbuf),
                                     scnt, posacc)

        sl = scnt & 127
        pb = sl - (sl & 15)
        tmp_pos[...] = posacc
        for g in range(8):
            idxv = g * L + _IOTA()
            cur = stag_pos[pl.ds(g * L, L)]
            rot = plsc.load_gather(tmp_pos, [(idxv - pb) & 15])
            stag_pos[pl.ds(g * L, L)] = jnp.where(
                idxv < pb, cur, jnp.where(idxv < sl, rot, _splat(dump)))
        pltpu.async_copy(stag, ord_hbm.at[stag_pos], sem_st).wait()

    return body


def _extract_kernel(n_cols, stripe, n_extra, tail_w):
    body = _make_extract(n_cols, stripe, n_extra, tail_w)
    mesh = plsc.VectorSubcoreMesh(core_axis_name="c", subcore_axis_name="s")
    return functools.partial(
        pl.kernel,
        mesh=mesh,
        out_type=jax.ShapeDtypeStruct((ORD_ROWS, 128), jnp.float32),
        scratch_types=[
            pltpu.VMEM((BATCH // 128, 128), jnp.int32),
            pltpu.VMEM((BATCH + 2 * L,), jnp.int32),
            pltpu.VMEM((BATCH + 2 * L,), jnp.int32),
            pltpu.VMEM((2, EMBED_DIM, CW), jnp.float32),
            pltpu.VMEM((128, 128), jnp.float32),
            pltpu.VMEM((128,), jnp.int32),
            pltpu.VMEM((L,), jnp.int32),
            pltpu.VMEM((L,), jnp.int32),
            pltpu.VMEM((EMBED_DIM, 128), jnp.float32),
            pltpu.VMEM((EMBED_DIM, tail_w), jnp.float32),
            pltpu.SemaphoreType.DMA,
            pltpu.SemaphoreType.DMA,
        ],
        compiler_params=pltpu.CompilerParams(needs_layout_passes=False),
    )(body)


def _dot_body(u_ord, a_ord, out_hbm, ub, ab, out_v, sem_u, sem_a):
    wid = lax.axis_index("s") * NC + lax.axis_index("c")
    base = wid * (BATCH // NW)
    for q in range(4):
        cu = pltpu.async_copy(u_ord.at[pl.ds(base + q * 128, 128)], ub, sem_u)
        ca = pltpu.async_copy(a_ord.at[pl.ds(base + q * 128, 128)], ab, sem_a)
        cu.wait()
        ca.wait()

        def group(g, carry, q=q):
            rows = g * L + _IOTA()
            acc = plsc.load_gather(ub, [rows, _splat(0)]) * \
                plsc.load_gather(ab, [rows, _splat(0)])
            for d in range(1, EMBED_DIM):
                acc = acc + plsc.load_gather(ub, [rows, _splat(d)]) * \
                    plsc.load_gather(ab, [rows, _splat(d)])
            out_v[pl.ds(q * 128 + g * L, L)] = acc
            return carry

        lax.fori_loop(0, 8, group, 0)
    pltpu.sync_copy(out_v, out_hbm.at[pl.ds(base, BATCH // NW)])


def _dot_kernel():
    mesh = plsc.VectorSubcoreMesh(core_axis_name="c", subcore_axis_name="s")
    return functools.partial(
        pl.kernel,
        mesh=mesh,
        out_type=jax.ShapeDtypeStruct((BATCH,), jnp.float32),
        scratch_types=[
            pltpu.VMEM((128, 128), jnp.float32),
            pltpu.VMEM((128, 128), jnp.float32),
            pltpu.VMEM((BATCH // NW,), jnp.float32),
            pltpu.SemaphoreType.DMA,
            pltpu.SemaphoreType.DMA,
        ],
        compiler_params=pltpu.CompilerParams(needs_layout_passes=False),
    )(_dot_body)


ART_STRIPE = 3072
ART_EXTRA = 13
ART_TAIL = 32
CUST_STRIPE = 31232
CUST_EXTRA = 4
CUST_TAIL = 64


@jax.jit
def _recommend_sc(user2d, article2d, cust_t, cust_tail, art_t, art_tail):
    a_ord = _extract_kernel(NUM_ARTICLES, ART_STRIPE, ART_EXTRA, ART_TAIL)(
        article2d, art_t, art_tail)
    u_ord = _extract_kernel(NUM_CUSTOMERS, CUST_STRIPE, CUST_EXTRA,
                            CUST_TAIL)(user2d, cust_t, cust_tail)
    return _dot_kernel()(u_ord, a_ord)


def kernel(user, article, customer_table, article_table):
    user2d = user.reshape(BATCH // 128, 128)
    article2d = article.reshape(BATCH // 128, 128)
    cust_t = customer_table.T
    art_t = article_table.T
    cust_tail = customer_table[NUM_CUSTOMERS - CUST_TAIL:].T
    art_tail = article_table[NUM_ARTICLES - ART_TAIL:].T
    scores = _recommend_sc(user2d, article2d, cust_t, cust_tail,
                           art_t, art_tail)
    return scores.reshape(BATCH, 1)


# speedup vs baseline: 1.6058x; 1.2055x over previous
import functools

import jax
import jax.numpy as jnp
from jax import lax
from jax.experimental import pallas as pl
from jax.experimental.pallas import tpu as pltpu
from jax.experimental.pallas import tpu_sc as plsc

NUM_CUSTOMERS = 1000000
NUM_ARTICLES = 100000
EMBED_DIM = 32
BATCH = 16384

NC, NS, L = 2, 16, 16
NW = NC * NS
CW = 512
ORD_ROWS = BATCH + 128
NVEC = BATCH // L

_IOTA = lambda: lax.iota(jnp.int32, L)


def _splat(x):
    return jnp.full((L,), x, jnp.int32)


def _take_ref(ref, i):
    return plsc.load_gather(ref, [_splat(i)])[0]


def _make_extract(n_cols, stripe, n_extra, tail_w):
    nchunk = stripe // CW
    extra0 = NW * stripe
    tail0 = extra0 + n_extra * 128
    assert tail0 + tail_w == n_cols

    def body(ids_hbm, table_hbm, tail_hbm, ord_hbm,
             ids_v, ml_id, ml_pos, ring, stag, stag_pos, tmp_id, tmp_pos,
             buf128, tailbuf, sem_ring, sem_st):
        wid = lax.axis_index("s") * NC + lax.axis_index("c")
        lo = wid * stripe
        hi = lo + stripe
        lo2 = jnp.where(wid < n_extra, extra0 + wid * 128,
                        jnp.where(wid == n_extra, tail0, -1))
        hi2 = jnp.where(wid < n_extra, extra0 + wid * 128 + 128,
                        jnp.where(wid == n_extra, tail0 + tail_w, -1))
        dump = BATCH + wid

        pltpu.sync_copy(ids_hbm, ids_v)
        pltpu.sync_copy(tail_hbm, tailbuf)
        pltpu.async_copy(table_hbm.at[:, pl.ds(lo, CW)], ring.at[0], sem_ring)
        pltpu.async_copy(table_hbm.at[:, pl.ds(lo + CW, CW)], ring.at[1],
                         sem_ring)

        def scan_vec(t, cnt):
            v = ids_v[t >> 3, pl.ds((t & 7) * L, L)]
            mask = ((v >= lo) & (v < hi)) | ((v >= lo2) & (v < hi2))
            k = plsc.all_reduce_population_count(mask)[0]

            def append(c):
                plsc.store_compressed(tmp_id.at[...], v, mask=mask)
                plsc.store_compressed(tmp_pos.at[...], t * L + _IOTA(), mask=mask)
                s0 = c & 15
                c16 = c - s0
                sh = (_IOTA() - s0) & 15
                keep = _IOTA() >= s0
                ml_id[pl.ds(c16, L)] = jnp.where(
                    keep, plsc.load_gather(tmp_id, [sh]),
                    ml_id[pl.ds(c16, L)])
                ml_pos[pl.ds(c16, L)] = jnp.where(
                    keep, plsc.load_gather(tmp_pos, [sh]),
                    ml_pos[pl.ds(c16, L)])
                sh2 = (_IOTA() + 16 - s0) & 15
                ml_id[pl.ds(c16 + L, L)] = plsc.load_gather(tmp_id, [sh2])
                ml_pos[pl.ds(c16 + L, L)] = plsc.load_gather(tmp_pos, [sh2])
                return c + k

            return lax.cond(k > 0, append, lambda c: c, cnt)

        cnt = lax.fori_loop(0, NVEC, scan_vec, 0)
        nv = (cnt + 15) >> 4

        def flush(scnt, posacc):
            s15 = scnt & 15

            @pl.when(s15 == 15)
            def _():
                stag_pos[pl.ds((scnt & 127) - 15, L)] = posacc

            @pl.when((scnt & 127) == 127)
            def _():
                pltpu.async_copy(stag, ord_hbm.at[stag_pos], sem_st).wait()

        def process_range(c0, width, loader, scnt, posacc):
            def per_vec(m, carry):
                scnt, posacc = carry
                mids = ml_id[pl.ds(m * L, L)]
                mpos = ml_pos[pl.ds(m * L, L)]
                valid = (m * L + _IOTA()) < cnt
                mask = jnp.where((mids >= c0) & (mids < c0 + width) & valid,
                                 1, 0)

                def cond(st):
                    return lax.reduce_sum_p.bind(st[0], axes=(0,)) > 0

                def step(st):
                    mask, scnt, posacc = st
                    l = plsc.all_reduce_ffs(mask > 0)[0]
                    cid = _take_ref(ml_id, m * L + l)
                    cpos = _take_ref(ml_pos, m * L + l)
                    g0, g1 = loader(cid - c0)
                    row = scnt & 127
                    stag[row, pl.ds(0, L)] = g0
                    stag[row, pl.ds(L, L)] = g1
                    posacc = jnp.where(_IOTA() == (scnt & 15), cpos, posacc)
                    flush(scnt, posacc)
                    return (jnp.where(_IOTA() == l, 0, mask), scnt + 1,
                            posacc)

                mask, scnt, posacc = lax.while_loop(
                    cond, step, (mask, scnt, posacc))
                return (scnt, posacc)

            return lax.fori_loop(0, nv, per_vec, (scnt, posacc))

        def ring_loader(slot):
            def load(col):
                g0 = plsc.load_gather(ring, [_splat(slot), _IOTA(),
                                             _splat(col)])
                g1 = plsc.load_gather(ring, [_splat(slot), _IOTA() + L,
                                             _splat(col)])
                return g0, g1
            return load

        def buf_loader(buf):
            def load(col):
                g0 = plsc.load_gather(buf, [_IOTA(), _splat(col)])
                g1 = plsc.load_gather(buf, [_IOTA() + L, _splat(col)])
                return g0, g1
            return load

        def chunk(j, carry):
            scnt, posacc = carry
            pltpu.make_async_copy(table_hbm.at[:, pl.ds(lo, CW)],
                                  ring.at[j & 1], sem_ring).wait()
            carry = process_range(lo + j * CW, CW, ring_loader(j & 1),
                                  scnt, posacc)

            @pl.when(j + 2 < nchunk)
            def _():
                pltpu.async_copy(table_hbm.at[:, pl.ds(lo + (j + 2) * CW, CW)],
                                 ring.at[j & 1], sem_ring)

            return carry

        scnt, posacc = lax.fori_loop(0, nchunk, chunk, (0, jnp.zeros(
            (L,), jnp.int32)))

        c128 = jnp.where(wid < n_extra, extra0 + wid * 128, 0)
        pltpu.sync_copy(table_hbm.at[:, pl.ds(c128, 128)], buf128)
        scnt, posacc = process_range(c128, jnp.where(wid < n_extra, 128, 0),
                                     buf_loader(buf128), scnt, posacc)
        scnt, posacc = process_range(tail0, tail_w, buf_loader(tailbuf),
                                     scnt, posacc)

        sl = scnt & 127
        pb = sl - (sl & 15)
        tmp_pos[...] = posacc
        for g in range(8):
            idxv = g * L + _IOTA()
            cur = stag_pos[pl.ds(g * L, L)]
            rot = plsc.load_gather(tmp_pos, [(idxv - pb) & 15])
            stag_pos[pl.ds(g * L, L)] = jnp.where(
                idxv < pb, cur, jnp.where(idxv < sl, rot, _splat(dump)))
        pltpu.async_copy(stag, ord_hbm.at[stag_pos], sem_st).wait()

    return body


def _extract_kernel(n_cols, stripe, n_extra, tail_w):
    body = _make_extract(n_cols, stripe, n_extra, tail_w)
    mesh = plsc.VectorSubcoreMesh(core_axis_name="c", subcore_axis_name="s")
    return functools.partial(
        pl.kernel,
        mesh=mesh,
        out_type=jax.ShapeDtypeStruct((ORD_ROWS, 128), jnp.float32),
        scratch_types=[
            pltpu.VMEM((BATCH // 128, 128), jnp.int32),
            pltpu.VMEM((BATCH + 2 * L,), jnp.int32),
            pltpu.VMEM((BATCH + 2 * L,), jnp.int32),
            pltpu.VMEM((2, EMBED_DIM, CW), jnp.float32),
            pltpu.VMEM((128, 128), jnp.float32),
            pltpu.VMEM((128,), jnp.int32),
            pltpu.VMEM((L,), jnp.int32),
            pltpu.VMEM((L,), jnp.int32),
            pltpu.VMEM((EMBED_DIM, 128), jnp.float32),
            pltpu.VMEM((EMBED_DIM, tail_w), jnp.float32),
            pltpu.SemaphoreType.DMA,
            pltpu.SemaphoreType.DMA,
        ],
        compiler_params=pltpu.CompilerParams(needs_layout_passes=False),
    )(body)


def _dot_body(u_ord, a_ord, out_hbm, ub, ab, out_v, sem_u, sem_a):
    wid = lax.axis_index("s") * NC + lax.axis_index("c")
    base = wid * (BATCH // NW)
    for q in range(4):
        cu = pltpu.async_copy(u_ord.at[pl.ds(base + q * 128, 128)], ub, sem_u)
        ca = pltpu.async_copy(a_ord.at[pl.ds(base + q * 128, 128)], ab, sem_a)
        cu.wait()
        ca.wait()

        def group(g, carry, q=q):
            rows = g * L + _IOTA()
            acc = plsc.load_gather(ub, [rows, _splat(0)]) * \
                plsc.load_gather(ab, [rows, _splat(0)])
            for d in range(1, EMBED_DIM):
                acc = acc + plsc.load_gather(ub, [rows, _splat(d)]) * \
                    plsc.load_gather(ab, [rows, _splat(d)])
            out_v[pl.ds(q * 128 + g * L, L)] = acc
            return carry

        lax.fori_loop(0, 8, group, 0)
    pltpu.sync_copy(out_v, out_hbm.at[pl.ds(base, BATCH // NW)])


def _dot_kernel():
    mesh = plsc.VectorSubcoreMesh(core_axis_name="c", subcore_axis_name="s")
    return functools.partial(
        pl.kernel,
        mesh=mesh,
        out_type=jax.ShapeDtypeStruct((BATCH,), jnp.float32),
        scratch_types=[
            pltpu.VMEM((128, 128), jnp.float32),
            pltpu.VMEM((128, 128), jnp.float32),
            pltpu.VMEM((BATCH // NW,), jnp.float32),
            pltpu.SemaphoreType.DMA,
            pltpu.SemaphoreType.DMA,
        ],
        compiler_params=pltpu.CompilerParams(needs_layout_passes=False),
    )(_dot_body)


ART_STRIPE = 3072
ART_EXTRA = 13
ART_TAIL = 32
CUST_STRIPE = 31232
CUST_EXTRA = 4
CUST_TAIL = 64


@jax.jit
def _recommend_sc(user2d, article2d, cust_t, cust_tail, art_t, art_tail):
    a_ord = _extract_kernel(NUM_ARTICLES, ART_STRIPE, ART_EXTRA, ART_TAIL)(
        article2d, art_t, art_tail)
    u_ord = _extract_kernel(NUM_CUSTOMERS, CUST_STRIPE, CUST_EXTRA,
                            CUST_TAIL)(user2d, cust_t, cust_tail)
    return _dot_kernel()(u_ord, a_ord)


def kernel(user, article, customer_table, article_table):
    user2d = user.reshape(BATCH // 128, 128)
    article2d = article.reshape(BATCH // 128, 128)
    cust_t = customer_table.T
    art_t = article_table.T
    cust_tail = customer_table[NUM_CUSTOMERS - CUST_TAIL:].T
    art_tail = article_table[NUM_ARTICLES - ART_TAIL:].T
    scores = _recommend_sc(user2d, article2d, cust_t, cust_tail,
                           art_t, art_tail)
    return scores.reshape(BATCH, 1)


# speedup vs baseline: 1.6723x; 1.0414x over previous
import functools

import jax
import jax.numpy as jnp
from jax import lax
from jax.experimental import pallas as pl
from jax.experimental.pallas import tpu as pltpu
from jax.experimental.pallas import tpu_sc as plsc

NUM_CUSTOMERS = 1000000
NUM_ARTICLES = 100000
EMBED_DIM = 32
BATCH = 16384

NC, NS, L = 2, 16, 16
NW = NC * NS
CW = 512
ORD_ROWS = BATCH + 128
NVEC = BATCH // L

_IOTA = lambda: lax.iota(jnp.int32, L)


def _splat(x):
    return jnp.full((L,), x, jnp.int32)


def _take_ref(ref, i):
    return plsc.load_gather(ref, [_splat(i)])[0]


def _make_extract(n_cols, stripe, n_extra, tail_w):
    nchunk = stripe // CW
    extra0 = NW * stripe
    tail0 = extra0 + n_extra * 128
    assert tail0 + tail_w == n_cols

    def body(ids_hbm, table_hbm, tail_hbm, ord_hbm,
             ids_v, ml_id, ml_pos, ring, stag, stag_pos, tmp_id, tmp_pos,
             buf128, tailbuf, sem_ring, sem_st):
        wid = lax.axis_index("s") * NC + lax.axis_index("c")
        lo = wid * stripe
        hi = lo + stripe
        lo2 = jnp.where(wid < n_extra, extra0 + wid * 128,
                        jnp.where(wid == n_extra, tail0, -1))
        hi2 = jnp.where(wid < n_extra, extra0 + wid * 128 + 128,
                        jnp.where(wid == n_extra, tail0 + tail_w, -1))
        dump = BATCH + wid

        pltpu.sync_copy(ids_hbm, ids_v)
        pltpu.sync_copy(tail_hbm, tailbuf)
        pltpu.async_copy(table_hbm.at[:, pl.ds(lo, CW)], ring.at[0], sem_ring)
        pltpu.async_copy(table_hbm.at[:, pl.ds(lo + CW, CW)], ring.at[1],
                         sem_ring)

        def scan_vec(t, cnt):
            v = ids_v[t >> 3, pl.ds((t & 7) * L, L)]
            mask = ((v >= lo) & (v < hi)) | ((v >= lo2) & (v < hi2))
            k = plsc.all_reduce_population_count(mask)[0]

            def append(c):
                plsc.store_compressed(tmp_id.at[...], v, mask=mask)
                plsc.store_compressed(tmp_pos.at[...], t * L + _IOTA(), mask=mask)
                s0 = c & 15
                c16 = c - s0
                sh = (_IOTA() - s0) & 15
                keep = _IOTA() >= s0
                ml_id[pl.ds(c16, L)] = jnp.where(
                    keep, plsc.load_gather(tmp_id, [sh]),
                    ml_id[pl.ds(c16, L)])
                ml_pos[pl.ds(c16, L)] = jnp.where(
                    keep, plsc.load_gather(tmp_pos, [sh]),
                    ml_pos[pl.ds(c16, L)])
                sh2 = (_IOTA() + 16 - s0) & 15
                ml_id[pl.ds(c16 + L, L)] = plsc.load_gather(tmp_id, [sh2])
                ml_pos[pl.ds(c16 + L, L)] = plsc.load_gather(tmp_pos, [sh2])
                return c + k

            return lax.cond(k > 0, append, lambda c: c, cnt)

        cnt = lax.fori_loop(0, NVEC, scan_vec, 0)
        nv = (cnt + 15) >> 4

        def flush(scnt, posacc):
            s15 = scnt & 15

            @pl.when(s15 == 15)
            def _():
                stag_pos[pl.ds((scnt & 127) - 15, L)] = posacc

            @pl.when((scnt & 127) == 127)
            def _():
                pltpu.async_copy(stag, ord_hbm.at[stag_pos], sem_st).wait()

        def process_range(c0, width, loader, scnt, posacc):
            def per_vec(m, carry):
                scnt, posacc = carry
                mids = ml_id[pl.ds(m * L, L)]
                mpos = ml_pos[pl.ds(m * L, L)]
                valid = (m * L + _IOTA()) < cnt
                mask = jnp.where((mids >= c0) & (mids < c0 + width) & valid,
                                 1, 0)

                def cond(st):
                    return plsc.all_reduce_population_count(st[0] > 0)[0] > 0

                def step(st):
                    mask, scnt, posacc = st
                    l = plsc.all_reduce_ffs(mask > 0)[0]
                    cid = _take_ref(ml_id, m * L + l)
                    cpos = _take_ref(ml_pos, m * L + l)
                    g0, g1 = loader(cid - c0)
                    row = scnt & 127
                    stag[row, pl.ds(0, L)] = g0
                    stag[row, pl.ds(L, L)] = g1
                    posacc = jnp.where(_IOTA() == (scnt & 15), cpos, posacc)
                    flush(scnt, posacc)
                    return (jnp.where(_IOTA() == l, 0, mask), scnt + 1,
                            posacc)

                mask, scnt, posacc = lax.while_loop(
                    cond, step, (mask, scnt, posacc))
                return (scnt, posacc)

            return lax.fori_loop(0, nv, per_vec, (scnt, posacc))

        def ring_loader(slot):
            def load(col):
                g0 = plsc.load_gather(ring, [_splat(slot), _IOTA(),
                                             _splat(col)])
                g1 = plsc.load_gather(ring, [_splat(slot), _IOTA() + L,
                                             _splat(col)])
                return g0, g1
            return load

        def buf_loader(buf):
            def load(col):
                g0 = plsc.load_gather(buf, [_IOTA(), _splat(col)])
                g1 = plsc.load_gather(buf, [_IOTA() + L, _splat(col)])
                return g0, g1
            return load

        def chunk(j, carry):
            scnt, posacc = carry
            pltpu.make_async_copy(table_hbm.at[:, pl.ds(lo, CW)],
                                  ring.at[j & 1], sem_ring).wait()
            carry = process_range(lo + j * CW, CW, ring_loader(j & 1),
                                  scnt, posacc)

            @pl.when(j + 2 < nchunk)
            def _():
                pltpu.async_copy(table_hbm.at[:, pl.ds(lo + (j + 2) * CW, CW)],
                                 ring.at[j & 1], sem_ring)

            return carry

        scnt, posacc = lax.fori_loop(0, nchunk, chunk, (0, jnp.zeros(
            (L,), jnp.int32)))

        c128 = jnp.where(wid < n_extra, extra0 + wid * 128, 0)
        pltpu.sync_copy(table_hbm.at[:, pl.ds(c128, 128)], buf128)
        scnt, posacc = process_range(c128, jnp.where(wid < n_extra, 128, 0),
                                     buf_loader(buf128), scnt, posacc)
        scnt, posacc = process_range(tail0, tail_w, buf_loader(tailbuf),
                                     scnt, posacc)

        sl = scnt & 127
        pb = sl - (sl & 15)
        tmp_pos[...] = posacc
        for g in range(8):
            idxv = g * L + _IOTA()
            cur = stag_pos[pl.ds(g * L, L)]
            rot = plsc.load_gather(tmp_pos, [(idxv - pb) & 15])
            stag_pos[pl.ds(g * L, L)] = jnp.where(
                idxv < pb, cur, jnp.where(idxv < sl, rot, _splat(dump)))
        pltpu.async_copy(stag, ord_hbm.at[stag_pos], sem_st).wait()

    return body


def _extract_kernel(n_cols, stripe, n_extra, tail_w):
    body = _make_extract(n_cols, stripe, n_extra, tail_w)
    mesh = plsc.VectorSubcoreMesh(core_axis_name="c", subcore_axis_name="s")
    return functools.partial(
        pl.kernel,
        mesh=mesh,
        out_type=jax.ShapeDtypeStruct((ORD_ROWS, 128), jnp.float32),
        scratch_types=[
            pltpu.VMEM((BATCH // 128, 128), jnp.int32),
            pltpu.VMEM((BATCH + 2 * L,), jnp.int32),
            pltpu.VMEM((BATCH + 2 * L,), jnp.int32),
            pltpu.VMEM((2, EMBED_DIM, CW), jnp.float32),
            pltpu.VMEM((128, 128), jnp.float32),
            pltpu.VMEM((128,), jnp.int32),
            pltpu.VMEM((L,), jnp.int32),
            pltpu.VMEM((L,), jnp.int32),
            pltpu.VMEM((EMBED_DIM, 128), jnp.float32),
            pltpu.VMEM((EMBED_DIM, tail_w), jnp.float32),
            pltpu.SemaphoreType.DMA,
            pltpu.SemaphoreType.DMA,
        ],
        compiler_params=pltpu.CompilerParams(needs_layout_passes=False),
    )(body)


def _dot_body(u_ord, a_ord, out_hbm, ub, ab, out_v, sem_u, sem_a):
    wid = lax.axis_index("s") * NC + lax.axis_index("c")
    base = wid * (BATCH // NW)
    for q in range(4):
        cu = pltpu.async_copy(u_ord.at[pl.ds(base + q * 128, 128)], ub, sem_u)
        ca = pltpu.async_copy(a_ord.at[pl.ds(base + q * 128, 128)], ab, sem_a)
        cu.wait()
        ca.wait()

        def group(g, carry, q=q):
            rows = g * L + _IOTA()
            acc = plsc.load_gather(ub, [rows, _splat(0)]) * \
                plsc.load_gather(ab, [rows, _splat(0)])
            for d in range(1, EMBED_DIM):
                acc = acc + plsc.load_gather(ub, [rows, _splat(d)]) * \
                    plsc.load_gather(ab, [rows, _splat(d)])
            out_v[pl.ds(q * 128 + g * L, L)] = acc
            return carry

        lax.fori_loop(0, 8, group, 0)
    pltpu.sync_copy(out_v, out_hbm.at[pl.ds(base, BATCH // NW)])


def _dot_kernel():
    mesh = plsc.VectorSubcoreMesh(core_axis_name="c", subcore_axis_name="s")
    return functools.partial(
        pl.kernel,
        mesh=mesh,
        out_type=jax.ShapeDtypeStruct((BATCH,), jnp.float32),
        scratch_types=[
            pltpu.VMEM((128, 128), jnp.float32),
            pltpu.VMEM((128, 128), jnp.float32),
            pltpu.VMEM((BATCH // NW,), jnp.float32),
            pltpu.SemaphoreType.DMA,
            pltpu.SemaphoreType.DMA,
        ],
        compiler_params=pltpu.CompilerParams(needs_layout_passes=False),
    )(_dot_body)


ART_STRIPE = 3072
ART_EXTRA = 13
ART_TAIL = 32
CUST_STRIPE = 31232
CUST_EXTRA = 4
CUST_TAIL = 64


@jax.jit
def _recommend_sc(user2d, article2d, cust_t, cust_tail, art_t, art_tail):
    a_ord = _extract_kernel(NUM_ARTICLES, ART_STRIPE, ART_EXTRA, ART_TAIL)(
        article2d, art_t, art_tail)
    u_ord = _extract_kernel(NUM_CUSTOMERS, CUST_STRIPE, CUST_EXTRA,
                            CUST_TAIL)(user2d, cust_t, cust_tail)
    return _dot_kernel()(u_ord, a_ord)


def kernel(user, article, customer_table, article_table):
    user2d = user.reshape(BATCH // 128, 128)
    article2d = article.reshape(BATCH // 128, 128)
    cust_t = customer_table.T
    art_t = article_table.T
    cust_tail = customer_table[NUM_CUSTOMERS - CUST_TAIL:].T
    art_tail = article_table[NUM_ARTICLES - ART_TAIL:].T
    scores = _recommend_sc(user2d, article2d, cust_t, cust_tail,
                           art_t, art_tail)
    return scores.reshape(BATCH, 1)
